# baseline (speedup 1.0000x reference)
import functools
import jax
import jax.numpy as jnp
from jax import lax
from jax.experimental import pallas as pl
from jax.experimental.pallas import tpu as pltpu
from jax.experimental.pallas import tpu_sc as plsc

N = 10000
NP = 10240
E = 320000
B = 2
NS = 16
NPT = NP // NS
KA = 128
NBUF = 4
NGRP = 40
EPTP = NGRP * NBUF * KA
EP = EPTP * NS
KD = 80
NBLKD = (E // NS) // KD
DW = 16
DEG_WIN = 8

_MESH = plsc.VectorSubcoreMesh(core_axis_name="c", subcore_axis_name="s")


NTT = N // NS


def _make_agg(D, DH):
  npass = D // DH

  @functools.partial(
      pl.kernel,
      out_type=jax.ShapeDtypeStruct((B, NP, D), jnp.float32),
      mesh=_MESH,
      scratch_types=(
          [pltpu.VMEM((NBUF, KA), jnp.int32) for _ in range(4)]
          + [pltpu.VMEM((KA, DH), jnp.float32) for _ in range(NBUF)]
          + [pltpu.SemaphoreType.DMA for _ in range(2 * NBUF + 4)]
          + [pltpu.VMEM_SHARED((N, DH), jnp.float32),
             pltpu.VMEM_SHARED((NP, DH), jnp.float32)]
      ),
      compiler_params=pltpu.CompilerParams(use_tc_tiling_on_sc=False),
  )
  def agg_kernel(t_hbm, src3_hbm, dst3_hbm, zeros_hbm, out_hbm, *scratch):
    sbuf = scratch[0:2]
    dbuf = scratch[2:4]
    rows = scratch[4:4 + NBUF]
    gsem = scratch[4 + NBUF:4 + 2 * NBUF]
    ssem = scratch[4 + 2 * NBUF:4 + 3 * NBUF]
    issem = scratch[4 + 3 * NBUF:6 + 3 * NBUF]
    idsem = scratch[6 + 3 * NBUF:8 + 3 * NBUF]
    tbl_sh, acc_sh = scratch[-2], scratch[-1]
    c = lax.axis_index("c")
    s = lax.axis_index("s")
    row0 = s * NPT
    trow0 = s * NTT

    def idx_start(p, g):
      pltpu.async_copy(src3_hbm.at[s, g], sbuf[p], issem[p])
      pltpu.async_copy(dst3_hbm.at[s, g], dbuf[p], idsem[p])

    def idx_wait(p, g):
      pltpu.make_async_copy(src3_hbm.at[s, g], sbuf[p], issem[p]).wait()
      pltpu.make_async_copy(dst3_hbm.at[s, g], dbuf[p], idsem[p]).wait()

    def gather_start(p, b):
      pltpu.async_copy(tbl_sh.at[sbuf[p].at[b]], rows[b], gsem[b])

    def gather_wait(p, b):
      pltpu.make_async_copy(tbl_sh.at[sbuf[p].at[b]], rows[b], gsem[b]).wait()

    def scat_start(p, b):
      pltpu.async_copy(rows[b], acc_sh.at[dbuf[p].at[b]], ssem[b], add=True)

    def scat_wait(p, b):
      pltpu.make_async_copy(rows[b], acc_sh.at[dbuf[p].at[b]], ssem[b]).wait()

    def process(g, p, q, prefetch_idx, start_next):
      for b in range(NBUF):
        gather_wait(p, b)
        scat_start(p, b)
      if start_next:
        idx_wait(q, g + 1)
      for b in range(NBUF):
        scat_wait(p, b)
        if start_next:
          gather_start(q, b)
      if prefetch_idx:
        idx_start(p, g + 2)

    for h in range(npass):
      pltpu.sync_copy(
          t_hbm.at[pl.ds(c * N + trow0, NTT), pl.ds(h * DH, DH)],
          tbl_sh.at[pl.ds(trow0, NTT), :])
      pltpu.sync_copy(zeros_hbm, acc_sh.at[pl.ds(row0, NPT), :])
      plsc.subcore_barrier()

      idx_start(0, 0)
      idx_wait(0, 0)
      for b in range(NBUF):
        gather_start(0, b)
      idx_start(1, 1)

      @pl.loop(0, (NGRP - 2) // 2)
      def _(i):
        g = 2 * i
        process(g, 0, 1, True, True)
        process(g + 1, 1, 0, True, True)

      process(NGRP - 2, 0, 1, False, True)
      process(NGRP - 1, 1, 0, False, False)

      plsc.subcore_barrier()
      pltpu.sync_copy(acc_sh.at[pl.ds(row0, NPT), :],
                      out_hbm.at[c, pl.ds(row0, NPT), pl.ds(h * DH, DH)])
      if h + 1 < npass:
        plsc.subcore_barrier()

  return agg_kernel


_agg128 = _make_agg(128, 64)
_agg16 = _make_agg(16, 16)


@functools.partial(
    pl.kernel,
    out_type=jax.ShapeDtypeStruct((B, NP, DW), jnp.float32),
    mesh=_MESH,
    scratch_types=[
        pltpu.VMEM((NBLKD, KD), jnp.int32),
        pltpu.VMEM((KD, DW), jnp.float32),
        pltpu.SemaphoreType.DMA,
        pltpu.VMEM_SHARED((NP, DW), jnp.float32),
    ],
    compiler_params=pltpu.CompilerParams(use_tc_tiling_on_sc=False),
)
def _deg_kernel(ei2_hbm, ones_hbm, zeros_hbm, out_hbm, idx_v, ones_v, sem,
                deg_sh):
  c = lax.axis_index("c")
  s = lax.axis_index("s")
  row0 = s * NPT
  pltpu.sync_copy(zeros_hbm, deg_sh.at[pl.ds(row0, NPT), :])
  pltpu.sync_copy(ei2_hbm.at[c, s], idx_v)
  pltpu.sync_copy(ones_hbm, ones_v)
  plsc.subcore_barrier()

  def scat_start(j):
    pltpu.async_copy(ones_v, deg_sh.at[idx_v.at[j]], sem, add=True)

  def scat_wait(j):
    pltpu.make_async_copy(ones_v, deg_sh.at[idx_v.at[j]], sem).wait()

  for j in range(DEG_WIN):
    scat_start(j)

  @pl.loop(0, NBLKD - DEG_WIN)
  def _(j):
    scat_start(j + DEG_WIN)
    scat_wait(j)

  for j in range(DEG_WIN):
    scat_wait(NBLKD - DEG_WIN + j)

  plsc.subcore_barrier()
  pltpu.sync_copy(deg_sh.at[pl.ds(row0, NPT), :],
                  out_hbm.at[c, pl.ds(row0, NPT), :])


_R = 2000


def _tc_first_body(x_ref, od_ref, w_ref, o_ref):
  od = od_ref[...]
  nsrc = jnp.where(od > 0.0, lax.rsqrt(od), 0.0)
  o_ref[...] = jnp.dot(x_ref[...] * nsrc, w_ref[...],
                       preferred_element_type=jnp.float32)


def _tc_first(x, od_col, w):
  m, din = x.shape
  dout = w.shape[1]
  return pl.pallas_call(
      _tc_first_body,
      grid=(m // _R,),
      in_specs=[
          pl.BlockSpec((_R, din), lambda i: (i, 0)),
          pl.BlockSpec((_R, 1), lambda i: (i, 0)),
          pl.BlockSpec((din, dout), lambda i: (0, 0)),
      ],
      out_specs=pl.BlockSpec((_R, dout), lambda i: (i, 0)),
      out_shape=jax.ShapeDtypeStruct((m, dout), jnp.float32),
  )(x, od_col, w)


def _tc_mid_body(a_ref, id_ref, od_ref, b_ref, w_ref, o_ref):
  idg = id_ref[...]
  od = od_ref[...]
  ndst = jnp.where(idg > 0.0, lax.rsqrt(idg), 0.0)
  nsrc = jnp.where(od > 0.0, lax.rsqrt(od), 0.0)
  h = jnp.maximum(a_ref[...] * ndst + b_ref[...], 0.0) * nsrc
  o_ref[...] = jnp.dot(h, w_ref[...], preferred_element_type=jnp.float32)


def _tc_mid(agg, id_col, od_col, b, w):
  m, din = agg.shape
  dout = w.shape[1]
  return pl.pallas_call(
      _tc_mid_body,
      grid=(m // _R,),
      in_specs=[
          pl.BlockSpec((_R, din), lambda i: (i, 0)),
          pl.BlockSpec((_R, 1), lambda i: (i, 0)),
          pl.BlockSpec((_R, 1), lambda i: (i, 0)),
          pl.BlockSpec((1, din), lambda i: (0, 0)),
          pl.BlockSpec((din, dout), lambda i: (0, 0)),
      ],
      out_specs=pl.BlockSpec((_R, dout), lambda i: (i, 0)),
      out_shape=jax.ShapeDtypeStruct((m, dout), jnp.float32),
  )(agg, id_col, od_col, b.reshape(1, din), w)


def _tc_final_body(a_ref, id_ref, b_ref, o_ref):
  idg = id_ref[...]
  ndst = jnp.where(idg > 0.0, lax.rsqrt(idg), 0.0)
  o_ref[...] = a_ref[...] * ndst + b_ref[...]


def _tc_final(agg, id_col, b):
  m, d = agg.shape
  return pl.pallas_call(
      _tc_final_body,
      grid=(m // _R,),
      in_specs=[
          pl.BlockSpec((_R, d), lambda i: (i, 0)),
          pl.BlockSpec((_R, 1), lambda i: (i, 0)),
          pl.BlockSpec((1, d), lambda i: (0, 0)),
      ],
      out_specs=pl.BlockSpec((_R, d), lambda i: (i, 0)),
      out_shape=jax.ShapeDtypeStruct((m, d), jnp.float32),
  )(agg, id_col, b.reshape(1, d))


def kernel(features, edge_index, W0, b0, W1, b1, W2, b2):
  src = edge_index[0].astype(jnp.int32)
  dst = edge_index[1].astype(jnp.int32)
  ei2 = jnp.stack([src, dst]).reshape(B, NS, NBLKD, KD)
  pad = EP - E
  pad_src = jnp.zeros((pad,), jnp.int32)
  pad_dst = jnp.full((pad,), N, jnp.int32)
  src3 = jnp.concatenate([src, pad_src]).reshape(NS, NGRP, NBUF, KA)
  dst3 = jnp.concatenate([dst, pad_dst]).reshape(NS, NGRP, NBUF, KA)

  x = features.reshape(B * N, features.shape[-1])
  zeros64 = jnp.zeros((NPT, 64), jnp.float32)
  zeros16 = jnp.zeros((NPT, DW), jnp.float32)
  ones_blk = jnp.ones((KD, DW), jnp.float32)

  degs = _deg_kernel(ei2, ones_blk, zeros16)
  od_col = jnp.tile(degs[0, :N, 0], B)[:, None]
  id_col = jnp.tile(degs[1, :N, 0], B)[:, None]

  t0 = _tc_first(x, od_col, W0)
  a0 = _agg128(t0, src3, dst3, zeros64)[:, :N].reshape(-1, 128)
  t1 = _tc_mid(a0, id_col, od_col, b0, W1)
  a1 = _agg128(t1, src3, dst3, zeros64)[:, :N].reshape(-1, 128)
  t2 = _tc_mid(a1, id_col, od_col, b1, W2)
  a2 = _agg16(t2, src3, dst3, zeros16)[:, :N].reshape(-1, 16)
  return _tc_final(a2, id_col, b2)


# speedup vs baseline: 1.0965x; 1.0965x over previous
import functools
import jax
import jax.numpy as jnp
from jax import lax
from jax.experimental import pallas as pl
from jax.experimental.pallas import tpu as pltpu
from jax.experimental.pallas import tpu_sc as plsc

N = 10000
NP = 10240
E = 320000
B = 2
NS = 16
NPT = NP // NS
KA = 64
NBUF = 8
NGRP = 40
EPTP = NGRP * NBUF * KA
EP = EPTP * NS
KD = 80
NBLKD = (E // NS) // KD
DW = 16
DEG_WIN = 8

_MESH = plsc.VectorSubcoreMesh(core_axis_name="c", subcore_axis_name="s")


NTT = N // NS


def _make_agg(D, DH):
  npass = D // DH

  @functools.partial(
      pl.kernel,
      out_type=jax.ShapeDtypeStruct((B, NP, D), jnp.float32),
      mesh=_MESH,
      scratch_types=(
          [pltpu.VMEM((NBUF, KA), jnp.int32) for _ in range(4)]
          + [pltpu.VMEM((KA, DH), jnp.float32) for _ in range(NBUF)]
          + [pltpu.SemaphoreType.DMA for _ in range(2 * NBUF + 4)]
          + [pltpu.VMEM_SHARED((N, DH), jnp.float32),
             pltpu.VMEM_SHARED((NP, DH), jnp.float32)]
      ),
      compiler_params=pltpu.CompilerParams(use_tc_tiling_on_sc=False),
  )
  def agg_kernel(t_hbm, src3_hbm, dst3_hbm, zeros_hbm, out_hbm, *scratch):
    sbuf = scratch[0:2]
    dbuf = scratch[2:4]
    rows = scratch[4:4 + NBUF]
    gsem = scratch[4 + NBUF:4 + 2 * NBUF]
    ssem = scratch[4 + 2 * NBUF:4 + 3 * NBUF]
    issem = scratch[4 + 3 * NBUF:6 + 3 * NBUF]
    idsem = scratch[6 + 3 * NBUF:8 + 3 * NBUF]
    tbl_sh, acc_sh = scratch[-2], scratch[-1]
    c = lax.axis_index("c")
    s = lax.axis_index("s")
    row0 = s * NPT
    trow0 = s * NTT

    def idx_start(p, g):
      pltpu.async_copy(src3_hbm.at[s, g], sbuf[p], issem[p])
      pltpu.async_copy(dst3_hbm.at[s, g], dbuf[p], idsem[p])

    def idx_wait(p, g):
      pltpu.make_async_copy(src3_hbm.at[s, g], sbuf[p], issem[p]).wait()
      pltpu.make_async_copy(dst3_hbm.at[s, g], dbuf[p], idsem[p]).wait()

    def gather_start(p, b):
      pltpu.async_copy(tbl_sh.at[sbuf[p].at[b]], rows[b], gsem[b])

    def gather_wait(p, b):
      pltpu.make_async_copy(tbl_sh.at[sbuf[p].at[b]], rows[b], gsem[b]).wait()

    def scat_start(p, b):
      pltpu.async_copy(rows[b], acc_sh.at[dbuf[p].at[b]], ssem[b], add=True)

    def scat_wait(p, b):
      pltpu.make_async_copy(rows[b], acc_sh.at[dbuf[p].at[b]], ssem[b]).wait()

    def process(g, p, q, prefetch_idx, start_next):
      for b in range(NBUF):
        gather_wait(p, b)
        scat_start(p, b)
      if start_next:
        idx_wait(q, g + 1)
      for b in range(NBUF):
        scat_wait(p, b)
        if start_next:
          gather_start(q, b)
      if prefetch_idx:
        idx_start(p, g + 2)

    for h in range(npass):
      pltpu.sync_copy(
          t_hbm.at[pl.ds(c * N + trow0, NTT), pl.ds(h * DH, DH)],
          tbl_sh.at[pl.ds(trow0, NTT), :])
      pltpu.sync_copy(zeros_hbm, acc_sh.at[pl.ds(row0, NPT), :])
      plsc.subcore_barrier()

      idx_start(0, 0)
      idx_wait(0, 0)
      for b in range(NBUF):
        gather_start(0, b)
      idx_start(1, 1)

      @pl.loop(0, (NGRP - 2) // 2)
      def _(i):
        g = 2 * i
        process(g, 0, 1, True, True)
        process(g + 1, 1, 0, True, True)

      process(NGRP - 2, 0, 1, False, True)
      process(NGRP - 1, 1, 0, False, False)

      plsc.subcore_barrier()
      pltpu.sync_copy(acc_sh.at[pl.ds(row0, NPT), :],
                      out_hbm.at[c, pl.ds(row0, NPT), pl.ds(h * DH, DH)])
      if h + 1 < npass:
        plsc.subcore_barrier()

  return agg_kernel


_agg128 = _make_agg(128, 64)
_agg16 = _make_agg(16, 16)


@functools.partial(
    pl.kernel,
    out_type=jax.ShapeDtypeStruct((B, NP, DW), jnp.float32),
    mesh=_MESH,
    scratch_types=[
        pltpu.VMEM((NBLKD, KD), jnp.int32),
        pltpu.VMEM((KD, DW), jnp.float32),
        pltpu.SemaphoreType.DMA,
        pltpu.VMEM_SHARED((NP, DW), jnp.float32),
    ],
    compiler_params=pltpu.CompilerParams(use_tc_tiling_on_sc=False),
)
def _deg_kernel(ei2_hbm, ones_hbm, zeros_hbm, out_hbm, idx_v, ones_v, sem,
                deg_sh):
  c = lax.axis_index("c")
  s = lax.axis_index("s")
  row0 = s * NPT
  pltpu.sync_copy(zeros_hbm, deg_sh.at[pl.ds(row0, NPT), :])
  pltpu.sync_copy(ei2_hbm.at[c, s], idx_v)
  pltpu.sync_copy(ones_hbm, ones_v)
  plsc.subcore_barrier()

  def scat_start(j):
    pltpu.async_copy(ones_v, deg_sh.at[idx_v.at[j]], sem, add=True)

  def scat_wait(j):
    pltpu.make_async_copy(ones_v, deg_sh.at[idx_v.at[j]], sem).wait()

  for j in range(DEG_WIN):
    scat_start(j)

  @pl.loop(0, NBLKD - DEG_WIN)
  def _(j):
    scat_start(j + DEG_WIN)
    scat_wait(j)

  for j in range(DEG_WIN):
    scat_wait(NBLKD - DEG_WIN + j)

  plsc.subcore_barrier()
  pltpu.sync_copy(deg_sh.at[pl.ds(row0, NPT), :],
                  out_hbm.at[c, pl.ds(row0, NPT), :])


_R = 2000


def _tc_first_body(x_ref, od_ref, w_ref, o_ref):
  od = od_ref[...]
  nsrc = jnp.where(od > 0.0, lax.rsqrt(od), 0.0)
  o_ref[...] = jnp.dot(x_ref[...] * nsrc, w_ref[...],
                       preferred_element_type=jnp.float32)


def _tc_first(x, od_col, w):
  m, din = x.shape
  dout = w.shape[1]
  return pl.pallas_call(
      _tc_first_body,
      grid=(m // _R,),
      in_specs=[
          pl.BlockSpec((_R, din), lambda i: (i, 0)),
          pl.BlockSpec((_R, 1), lambda i: (i, 0)),
          pl.BlockSpec((din, dout), lambda i: (0, 0)),
      ],
      out_specs=pl.BlockSpec((_R, dout), lambda i: (i, 0)),
      out_shape=jax.ShapeDtypeStruct((m, dout), jnp.float32),
  )(x, od_col, w)


def _tc_mid_body(a_ref, id_ref, od_ref, b_ref, w_ref, o_ref):
  idg = id_ref[...]
  od = od_ref[...]
  ndst = jnp.where(idg > 0.0, lax.rsqrt(idg), 0.0)
  nsrc = jnp.where(od > 0.0, lax.rsqrt(od), 0.0)
  h = jnp.maximum(a_ref[...] * ndst + b_ref[...], 0.0) * nsrc
  o_ref[...] = jnp.dot(h, w_ref[...], preferred_element_type=jnp.float32)


def _tc_mid(agg, id_col, od_col, b, w):
  m, din = agg.shape
  dout = w.shape[1]
  return pl.pallas_call(
      _tc_mid_body,
      grid=(m // _R,),
      in_specs=[
          pl.BlockSpec((_R, din), lambda i: (i, 0)),
          pl.BlockSpec((_R, 1), lambda i: (i, 0)),
          pl.BlockSpec((_R, 1), lambda i: (i, 0)),
          pl.BlockSpec((1, din), lambda i: (0, 0)),
          pl.BlockSpec((din, dout), lambda i: (0, 0)),
      ],
      out_specs=pl.BlockSpec((_R, dout), lambda i: (i, 0)),
      out_shape=jax.ShapeDtypeStruct((m, dout), jnp.float32),
  )(agg, id_col, od_col, b.reshape(1, din), w)


def _tc_final_body(a_ref, id_ref, b_ref, o_ref):
  idg = id_ref[...]
  ndst = jnp.where(idg > 0.0, lax.rsqrt(idg), 0.0)
  o_ref[...] = a_ref[...] * ndst + b_ref[...]


def _tc_final(agg, id_col, b):
  m, d = agg.shape
  return pl.pallas_call(
      _tc_final_body,
      grid=(m // _R,),
      in_specs=[
          pl.BlockSpec((_R, d), lambda i: (i, 0)),
          pl.BlockSpec((_R, 1), lambda i: (i, 0)),
          pl.BlockSpec((1, d), lambda i: (0, 0)),
      ],
      out_specs=pl.BlockSpec((_R, d), lambda i: (i, 0)),
      out_shape=jax.ShapeDtypeStruct((m, d), jnp.float32),
  )(agg, id_col, b.reshape(1, d))


def kernel(features, edge_index, W0, b0, W1, b1, W2, b2):
  src = edge_index[0].astype(jnp.int32)
  dst = edge_index[1].astype(jnp.int32)
  ei2 = jnp.stack([src, dst]).reshape(B, NS, NBLKD, KD)
  pad = EP - E
  pad_src = jnp.zeros((pad,), jnp.int32)
  pad_dst = jnp.full((pad,), N, jnp.int32)
  src3 = jnp.concatenate([src, pad_src]).reshape(NS, NGRP, NBUF, KA)
  dst3 = jnp.concatenate([dst, pad_dst]).reshape(NS, NGRP, NBUF, KA)

  x = features.reshape(B * N, features.shape[-1])
  zeros64 = jnp.zeros((NPT, 64), jnp.float32)
  zeros16 = jnp.zeros((NPT, DW), jnp.float32)
  ones_blk = jnp.ones((KD, DW), jnp.float32)

  degs = _deg_kernel(ei2, ones_blk, zeros16)
  od_col = jnp.tile(degs[0, :N, 0], B)[:, None]
  id_col = jnp.tile(degs[1, :N, 0], B)[:, None]

  t0 = _tc_first(x, od_col, W0)
  a0 = _agg128(t0, src3, dst3, zeros64)[:, :N].reshape(-1, 128)
  t1 = _tc_mid(a0, id_col, od_col, b0, W1)
  a1 = _agg128(t1, src3, dst3, zeros64)[:, :N].reshape(-1, 128)
  t2 = _tc_mid(a1, id_col, od_col, b1, W2)
  a2 = _agg16(t2, src3, dst3, zeros16)[:, :N].reshape(-1, 16)
  return _tc_final(a2, id_col, b2)


# speedup vs baseline: 1.1353x; 1.0354x over previous
import functools
import jax
import jax.numpy as jnp
from jax import lax
from jax.experimental import pallas as pl
from jax.experimental.pallas import tpu as pltpu
from jax.experimental.pallas import tpu_sc as plsc

N = 10000
NP = 10240
E = 320000
B = 2
NS = 16
NPT = NP // NS
KA = 64
NBUF = 8
NGRP = 40
EPTP = NGRP * NBUF * KA
EP = EPTP * NS
KD = 80
NBLKD = (E // NS) // KD
DW = 16
DEG_WIN = 8

_MESH = plsc.VectorSubcoreMesh(core_axis_name="c", subcore_axis_name="s")


NTT = N // NS


def _make_agg(D, DH):
  npass = D // DH

  @functools.partial(
      pl.kernel,
      out_type=jax.ShapeDtypeStruct((B, NP, D), jnp.float32),
      mesh=_MESH,
      scratch_types=(
          [pltpu.VMEM((NBUF, KA), jnp.int32) for _ in range(4)]
          + [pltpu.VMEM((KA, DH), jnp.float32) for _ in range(NBUF)]
          + [pltpu.SemaphoreType.DMA for _ in range(2 * NBUF + 4)]
          + [pltpu.VMEM_SHARED((N, DH), jnp.float32),
             pltpu.VMEM_SHARED((NP, DH), jnp.float32)]
      ),
      compiler_params=pltpu.CompilerParams(use_tc_tiling_on_sc=False),
  )
  def agg_kernel(t_hbm, src3_hbm, dst3_hbm, zeros_hbm, out_hbm, *scratch):
    sbuf = scratch[0:2]
    dbuf = scratch[2:4]
    rows = scratch[4:4 + NBUF]
    gsem = scratch[4 + NBUF:4 + 2 * NBUF]
    ssem = scratch[4 + 2 * NBUF:4 + 3 * NBUF]
    issem = scratch[4 + 3 * NBUF:6 + 3 * NBUF]
    idsem = scratch[6 + 3 * NBUF:8 + 3 * NBUF]
    tbl_sh, acc_sh = scratch[-2], scratch[-1]
    c = lax.axis_index("c")
    s = lax.axis_index("s")
    row0 = s * NPT
    trow0 = s * NTT

    def idx_start(p, g):
      pltpu.async_copy(src3_hbm.at[s, g], sbuf[p], issem[p])
      pltpu.async_copy(dst3_hbm.at[s, g], dbuf[p], idsem[p])

    def idx_wait(p, g):
      pltpu.make_async_copy(src3_hbm.at[s, g], sbuf[p], issem[p]).wait()
      pltpu.make_async_copy(dst3_hbm.at[s, g], dbuf[p], idsem[p]).wait()

    def gather_start(p, b):
      pltpu.async_copy(tbl_sh.at[sbuf[p].at[b]], rows[b], gsem[b])

    def gather_wait(p, b):
      pltpu.make_async_copy(tbl_sh.at[sbuf[p].at[b]], rows[b], gsem[b]).wait()

    def scat_start(p, b):
      pltpu.async_copy(rows[b], acc_sh.at[dbuf[p].at[b]], ssem[b], add=True)

    def scat_wait(p, b):
      pltpu.make_async_copy(rows[b], acc_sh.at[dbuf[p].at[b]], ssem[b]).wait()

    def process(g, p, q, prefetch_idx, start_next):
      for b in range(NBUF):
        gather_wait(p, b)
        scat_start(p, b)
      if start_next:
        idx_wait(q, g + 1)
      for b in range(NBUF):
        scat_wait(p, b)
        if start_next:
          gather_start(q, b)
      if prefetch_idx:
        idx_start(p, g + 2)

    for h in range(npass):
      pltpu.sync_copy(
          t_hbm.at[pl.ds(c * N + trow0, NTT), pl.ds(h * DH, DH)],
          tbl_sh.at[pl.ds(trow0, NTT), :])
      pltpu.sync_copy(zeros_hbm, acc_sh.at[pl.ds(row0, NPT), :])
      plsc.subcore_barrier()

      idx_start(0, 0)
      idx_wait(0, 0)
      for b in range(NBUF):
        gather_start(0, b)
      idx_start(1, 1)

      @pl.loop(0, (NGRP - 2) // 2)
      def _(i):
        g = 2 * i
        process(g, 0, 1, True, True)
        process(g + 1, 1, 0, True, True)

      process(NGRP - 2, 0, 1, False, True)
      process(NGRP - 1, 1, 0, False, False)

      plsc.subcore_barrier()
      pltpu.sync_copy(acc_sh.at[pl.ds(row0, NPT), :],
                      out_hbm.at[c, pl.ds(row0, NPT), pl.ds(h * DH, DH)])
      if h + 1 < npass:
        plsc.subcore_barrier()

  return agg_kernel


_agg128 = _make_agg(128, 64)
_agg16 = _make_agg(16, 16)


@functools.partial(
    pl.kernel,
    out_type=jax.ShapeDtypeStruct((B, NP, DW), jnp.float32),
    mesh=_MESH,
    scratch_types=[
        pltpu.VMEM((NBLKD, KD), jnp.int32),
        pltpu.VMEM((KD, DW), jnp.float32),
        pltpu.SemaphoreType.DMA,
        pltpu.VMEM_SHARED((NP, DW), jnp.float32),
    ],
    compiler_params=pltpu.CompilerParams(use_tc_tiling_on_sc=False),
)
def _deg_kernel(src2_hbm, dst2_hbm, ones_hbm, zeros_hbm, out_hbm, idx_v,
                ones_v, sem, deg_sh):
  c = lax.axis_index("c")
  s = lax.axis_index("s")
  row0 = s * NPT
  pltpu.sync_copy(zeros_hbm, deg_sh.at[pl.ds(row0, NPT), :])

  @pl.when(c == 0)
  def _():
    pltpu.sync_copy(src2_hbm.at[s], idx_v)

  @pl.when(c == 1)
  def _():
    pltpu.sync_copy(dst2_hbm.at[s], idx_v)

  pltpu.sync_copy(ones_hbm, ones_v)
  plsc.subcore_barrier()

  def scat_start(j):
    pltpu.async_copy(ones_v, deg_sh.at[idx_v.at[j]], sem, add=True)

  def scat_wait(j):
    pltpu.make_async_copy(ones_v, deg_sh.at[idx_v.at[j]], sem).wait()

  for j in range(DEG_WIN):
    scat_start(j)

  @pl.loop(0, NBLKD - DEG_WIN)
  def _(j):
    scat_start(j + DEG_WIN)
    scat_wait(j)

  for j in range(DEG_WIN):
    scat_wait(NBLKD - DEG_WIN + j)

  plsc.subcore_barrier()
  pltpu.sync_copy(deg_sh.at[pl.ds(row0, NPT), :],
                  out_hbm.at[c, pl.ds(row0, NPT), :])


_R = 2000
_NB = N // _R


def _tc_first_body(x_ref, deg_ref, w_ref, o_ref):
  od = deg_ref[0, :, :1]
  nsrc = jnp.where(od > 0.0, lax.rsqrt(od), 0.0)
  o_ref[...] = jnp.dot(x_ref[0] * nsrc, w_ref[...],
                       preferred_element_type=jnp.float32)


def _tc_first(x3, degs, w):
  din = x3.shape[-1]
  dout = w.shape[1]
  return pl.pallas_call(
      _tc_first_body,
      grid=(B * _NB,),
      in_specs=[
          pl.BlockSpec((1, _R, din), lambda i: (i // _NB, i % _NB, 0)),
          pl.BlockSpec((1, _R, DW), lambda i: (0, i % _NB, 0)),
          pl.BlockSpec((din, dout), lambda i: (0, 0)),
      ],
      out_specs=pl.BlockSpec((_R, dout), lambda i: (i, 0)),
      out_shape=jax.ShapeDtypeStruct((B * N, dout), jnp.float32),
  )(x3, degs, w)


def _tc_mid_body(a_ref, id_ref, od_ref, b_ref, w_ref, o_ref):
  idg = id_ref[0, :, :1]
  od = od_ref[0, :, :1]
  ndst = jnp.where(idg > 0.0, lax.rsqrt(idg), 0.0)
  nsrc = jnp.where(od > 0.0, lax.rsqrt(od), 0.0)
  h = jnp.maximum(a_ref[0] * ndst + b_ref[...], 0.0) * nsrc
  o_ref[...] = jnp.dot(h, w_ref[...], preferred_element_type=jnp.float32)


def _tc_mid(agg3, degs, b, w):
  din = agg3.shape[-1]
  dout = w.shape[1]
  return pl.pallas_call(
      _tc_mid_body,
      grid=(B * _NB,),
      in_specs=[
          pl.BlockSpec((1, _R, din), lambda i: (i // _NB, i % _NB, 0)),
          pl.BlockSpec((1, _R, DW), lambda i: (1, i % _NB, 0)),
          pl.BlockSpec((1, _R, DW), lambda i: (0, i % _NB, 0)),
          pl.BlockSpec((1, din), lambda i: (0, 0)),
          pl.BlockSpec((din, dout), lambda i: (0, 0)),
      ],
      out_specs=pl.BlockSpec((_R, dout), lambda i: (i, 0)),
      out_shape=jax.ShapeDtypeStruct((B * N, dout), jnp.float32),
  )(agg3, degs, degs, b.reshape(1, din), w)


def _tc_final_body(a_ref, id_ref, b_ref, o_ref):
  idg = id_ref[0, :, :1]
  ndst = jnp.where(idg > 0.0, lax.rsqrt(idg), 0.0)
  o_ref[...] = a_ref[0] * ndst + b_ref[...]


def _tc_final(agg3, degs, b):
  d = agg3.shape[-1]
  return pl.pallas_call(
      _tc_final_body,
      grid=(B * _NB,),
      in_specs=[
          pl.BlockSpec((1, _R, d), lambda i: (i // _NB, i % _NB, 0)),
          pl.BlockSpec((1, _R, DW), lambda i: (1, i % _NB, 0)),
          pl.BlockSpec((1, d), lambda i: (0, 0)),
      ],
      out_specs=pl.BlockSpec((_R, d), lambda i: (i, 0)),
      out_shape=jax.ShapeDtypeStruct((B * N, d), jnp.float32),
  )(agg3, degs, b.reshape(1, d))


def kernel(features, edge_index, W0, b0, W1, b1, W2, b2):
  src = edge_index[0].astype(jnp.int32)
  dst = edge_index[1].astype(jnp.int32)
  src2 = src.reshape(NS, NBLKD, KD)
  dst2 = dst.reshape(NS, NBLKD, KD)
  pad = EP - E
  src3 = jnp.concatenate([src, jnp.zeros((pad,), jnp.int32)]
                         ).reshape(NS, NGRP, NBUF, KA)
  dst3 = jnp.concatenate([dst, jnp.full((pad,), N, jnp.int32)]
                         ).reshape(NS, NGRP, NBUF, KA)

  zeros64 = jnp.zeros((NPT, 64), jnp.float32)
  zeros16 = jnp.zeros((NPT, DW), jnp.float32)
  ones_blk = jnp.ones((KD, DW), jnp.float32)

  degs = _deg_kernel(src2, dst2, ones_blk, zeros16)

  t0 = _tc_first(features, degs, W0)
  a0 = _agg128(t0, src3, dst3, zeros64)
  t1 = _tc_mid(a0, degs, b0, W1)
  a1 = _agg128(t1, src3, dst3, zeros64)
  t2 = _tc_mid(a1, degs, b1, W2)
  a2 = _agg16(t2, src3, dst3, zeros16)
  return _tc_final(a2, degs, b2)
